# Initial kernel scaffold; baseline (speedup 1.0000x reference)
import jax
import jax.numpy as jnp
from jax.experimental import pallas as pl


def kernel(x_path, x_link, x_node, params, e_pu, e_ip, e_cn, e_hl):
    raise NotImplementedError("write your pallas kernel here")


# speedup vs baseline: 4.5920x; 4.5920x over previous
import functools

import jax
import jax.numpy as jnp
from jax import lax
from jax.experimental import pallas as pl
from jax.experimental.pallas import tpu as pltpu
from jax.experimental.pallas import tpu_sc as plsc

N = 10000
D = 128
E = 320000
NC = 2
NS = 16
NW = NC * NS
EPW = E // NW
CHUNK = 80
NCH = EPW // CHUNK
RPT = 632
NPAD = NS * RPT


def _make_sc_agg(num_tables, table_ids):
    R = len(table_ids)
    mesh = plsc.VectorSubcoreMesh(core_axis_name="c", subcore_axis_name="s")

    @functools.partial(
        pl.kernel,
        out_type=[jax.ShapeDtypeStruct((NC, NPAD, D), jnp.float32)] * R,
        mesh=mesh,
        scratch_types=[
            pltpu.VMEM_SHARED((NPAD, D), jnp.float32),
            pltpu.VMEM((NCH, CHUNK), jnp.int32),
            pltpu.VMEM((NCH, CHUNK), jnp.int32),
            pltpu.VMEM((CHUNK, D), jnp.float32),
            pltpu.SemaphoreType.DMA,
        ],
    )
    def sc_agg(*args):
        tables = args[:num_tables]
        srcs, dsts, zeros_h = args[num_tables:num_tables + 3]
        outs = args[num_tables + 3:num_tables + 3 + R]
        acc, src_v, dst_v, rows, gsem = args[num_tables + 3 + R:]

        cid = lax.axis_index("c")
        sid = lax.axis_index("s")
        wid = sid * NC + cid

        for r, tid in enumerate(table_ids):
            table = tables[tid]
            out = outs[r]
            pltpu.sync_copy(srcs.at[r, wid], src_v)
            pltpu.sync_copy(dsts.at[r, wid], dst_v)
            pltpu.sync_copy(zeros_h.at[pl.ds(sid * RPT, RPT)],
                            acc.at[pl.ds(sid * RPT, RPT)])
            plsc.subcore_barrier()

            def body(j, carry):
                pltpu.async_copy(table.at[src_v.at[j]], rows, gsem).wait()
                pltpu.sync_copy(rows, acc.at[dst_v.at[j]], add=True)
                return carry

            lax.fori_loop(0, NCH, body, 0, unroll=False)
            plsc.subcore_barrier()
            pltpu.sync_copy(acc.at[pl.ds(sid * RPT, RPT)],
                            out.at[cid, pl.ds(sid * RPT, RPT)])
            plsc.subcore_barrier()

    return sc_agg


_sc_agg_l0 = _make_sc_agg(3, (0, 1, 2, 2))
_sc_agg_l1 = _make_sc_agg(2, (0, 1))


def _rsqrt(u):
    r = lax.rsqrt(u)
    r = r * (1.5 - 0.5 * u * r * r)
    r = r * (1.5 - 0.5 * u * r * r)
    return r


def _dot(a, b):
    return jnp.dot(a, b, preferred_element_type=jnp.float32)


def _gin(agg_ref, x, eps, W1, b1, g1, be1, W2, b2, concat):
    agg = agg_ref[0, :N] + agg_ref[1, :N]
    scale = 1.0 + eps[0, 0]
    if concat:
        out = jnp.concatenate([agg, scale * x], axis=1)
    else:
        out = agg + scale * x
    t = _dot(out, W1[...]) + b1[...]
    m = jnp.mean(t, axis=0, keepdims=True)
    c = t - m
    v = jnp.mean(c * c, axis=0, keepdims=True)
    h = c * _rsqrt(v + 1e-5) * g1[...] + be1[...]
    h = jnp.maximum(h, 0.0)
    return _dot(h, W2[...]) + b2[...]


def _tc_gin_body(cat, agg, x_ref, eps, W1, b1, g1, be1, W2, b2, o_ref):
    o_ref[...] = _gin(agg, x_ref[...], eps, W1, b1, g1, be1, W2, b2, cat)


def _tc_gin_addrelu_body(cat, agg, x_ref, prev_ref, eps, W1, b1, g1, be1, W2,
                         b2, o_ref):
    h = _gin(agg, x_ref[...], eps, W1, b1, g1, be1, W2, b2, cat)
    o_ref[...] = jnp.maximum(h + prev_ref[...], 0.0)


def _tc_gin_relu_body(cat, agg, x_ref, eps, W1, b1, g1, be1, W2, b2, o_ref):
    o_ref[...] = jnp.maximum(
        _gin(agg, x_ref[...], eps, W1, b1, g1, be1, W2, b2, cat), 0.0)


def _tc_final_body(agg, x_ref, prev_ref, eps, W1, b1, g1, be1, W2, b2,
                   Wp1, bp1, gp, bep, ap, Wp2, bp2, o_ref):
    h = _gin(agg, x_ref[...], eps, W1, b1, g1, be1, W2, b2, False)
    x2 = jnp.maximum(h + prev_ref[...], 0.0)
    t = _dot(x2, Wp1[...]) + bp1[...]
    m = jnp.mean(t, axis=0, keepdims=True)
    c = t - m
    v = jnp.mean(c * c, axis=0, keepdims=True)
    hb = c * _rsqrt(v + 1e-5) * gp[...] + bep[...]
    a = ap[0, 0]
    hb = jnp.where(hb >= 0, hb, a * hb)
    o = _dot(hb, Wp2[...]) + bp2[...]
    o_ref[...] = jnp.maximum(o, 0.0)


def _f32(shape):
    return jax.ShapeDtypeStruct(shape, jnp.float32)


_tc_gin_cat = pl.pallas_call(functools.partial(_tc_gin_body, True),
                             out_shape=_f32((N, D)))
_tc_gin_addrelu_cat = pl.pallas_call(
    functools.partial(_tc_gin_addrelu_body, True), out_shape=_f32((N, D)))
_tc_gin_relu_cat = pl.pallas_call(functools.partial(_tc_gin_relu_body, True),
                                  out_shape=_f32((N, D)))
_tc_gin_sum = pl.pallas_call(functools.partial(_tc_gin_body, False),
                             out_shape=_f32((N, D)))
_tc_final = pl.pallas_call(_tc_final_body, out_shape=_f32((N, 1)))


def _prep(p):
    return (p["eps"].reshape(1, 1), p["W1"], p["b1"].reshape(1, -1),
            p["g1"].reshape(1, -1), p["be1"].reshape(1, -1), p["W2"],
            p["b2"].reshape(1, -1))


def kernel(x_path, x_link, x_node, params, e_pu, e_ip, e_cn, e_hl):
    zeros = jnp.zeros((NPAD, D), jnp.float32)
    srcs = jnp.stack([e_pu[0], e_hl[0], e_ip[0], e_cn[0]]).reshape(
        4, NW, NCH, CHUNK)
    dsts = jnp.stack([e_pu[1], e_hl[1], e_ip[1], e_cn[1]]).reshape(
        4, NW, NCH, CHUNK)

    agg_pu, agg_hl, agg_ip, agg_cn = _sc_agg_l0(
        x_path, x_node, x_link, srcs, dsts, zeros)

    l0 = params["layer0"]
    x_path1 = _tc_gin_relu_cat(agg_ip, x_path, *_prep(l0["ip"]))
    x_node1 = _tc_gin_relu_cat(agg_cn, x_node, *_prep(l0["cn"]))
    t_pu = _tc_gin_cat(agg_pu, x_link, *_prep(l0["pu"]))
    x_link1 = _tc_gin_addrelu_cat(agg_hl, x_link, t_pu, *_prep(l0["hl"]))

    agg2_pu, agg2_hl = _sc_agg_l1(x_path1, x_node1, srcs[:2], dsts[:2], zeros)

    l1 = params["layer1"]
    pp = params["post"]
    t2_pu = _tc_gin_sum(agg2_pu, x_link1, *_prep(l1["pu"]))
    return _tc_final(agg2_hl, x_link1, t2_pu, *_prep(l1["hl"]),
                     pp["W1"], pp["b1"].reshape(1, -1), pp["g"].reshape(1, -1),
                     pp["be"].reshape(1, -1), pp["a"].reshape(1, 1),
                     pp["W2"], pp["b2"].reshape(1, 1))


# speedup vs baseline: 7.4975x; 1.6327x over previous
import functools

import jax
import jax.numpy as jnp
from jax import lax
from jax.experimental import pallas as pl
from jax.experimental.pallas import tpu as pltpu
from jax.experimental.pallas import tpu_sc as plsc

N = 10000
D = 128
E = 320000
NC = 2
NS = 16
NW = NC * NS
EPW = E // NW
CHUNK = 80
NCH = EPW // CHUNK
RPT = 632
NPAD = NS * RPT


def _make_sc_agg(num_tables, table_ids):
    R = len(table_ids)
    mesh = plsc.VectorSubcoreMesh(core_axis_name="c", subcore_axis_name="s")

    @functools.partial(
        pl.kernel,
        out_type=[jax.ShapeDtypeStruct((NC, NPAD, D), jnp.float32)] * R,
        mesh=mesh,
        scratch_types=[
            pltpu.VMEM_SHARED((NPAD, D), jnp.float32),
            pltpu.VMEM((EPW,), jnp.int32),
            pltpu.VMEM((NCH, CHUNK), jnp.int32),
            pltpu.VMEM((2, CHUNK, D), jnp.float32),
            pltpu.SemaphoreType.DMA((2,)),
        ],
    )
    def sc_agg(*args):
        tables = args[:num_tables]
        srcs, dsts, zeros_h = args[num_tables:num_tables + 3]
        outs = args[num_tables + 3:num_tables + 3 + R]
        acc, src_v, dst_v, rows, gsem = args[num_tables + 3 + R:]

        cid = lax.axis_index("c")
        sid = lax.axis_index("s")
        wid = sid * NC + cid

        for r, tid in enumerate(table_ids):
            table = tables[tid]
            out = outs[r]
            pltpu.sync_copy(srcs.at[r, wid], src_v)
            pltpu.sync_copy(dsts.at[r, wid], dst_v)
            pltpu.sync_copy(zeros_h.at[pl.ds(sid * RPT, RPT)],
                            acc.at[pl.ds(sid * RPT, RPT)])
            plsc.subcore_barrier()

            for b in range(2):
                pltpu.async_copy(table.at[src_v.at[pl.ds(b * CHUNK, CHUNK)]], rows.at[b],
                                 gsem.at[b])

            def group(g, carry):
                for b in range(2):
                    j = g * 2 + b

                    @pl.when(j < NCH)
                    def _():
                        pltpu.make_async_copy(
                            table.at[src_v.at[pl.ds(j * CHUNK, CHUNK)]],
                            rows.at[b], gsem.at[b]).wait()
                        pltpu.sync_copy(rows.at[b], acc.at[dst_v.at[j]],
                                        add=True)

                        @pl.when(j + 2 < NCH)
                        def _():
                            pltpu.async_copy(
                                table.at[src_v.at[pl.ds((j + 2) * CHUNK,
                                                        CHUNK)]],
                                rows.at[b], gsem.at[b])
                return carry

            lax.fori_loop(0, NCH // 2 + 1, group, 0, unroll=False)
            plsc.subcore_barrier()
            pltpu.sync_copy(acc.at[pl.ds(sid * RPT, RPT)],
                            out.at[cid, pl.ds(sid * RPT, RPT)])
            plsc.subcore_barrier()

    return sc_agg


_sc_agg_l0 = _make_sc_agg(3, (0, 1, 2, 2))
_sc_agg_l1 = _make_sc_agg(2, (0, 1))


def _rsqrt(u):
    r = lax.rsqrt(u)
    r = r * (1.5 - 0.5 * u * r * r)
    r = r * (1.5 - 0.5 * u * r * r)
    return r


def _dot(a, b):
    return jnp.dot(a, b, preferred_element_type=jnp.float32)


def _gin(agg_ref, x, eps, W1, b1, g1, be1, W2, b2, concat):
    agg = agg_ref[0, :N] + agg_ref[1, :N]
    scale = 1.0 + eps[0, 0]
    if concat:
        out = jnp.concatenate([agg, scale * x], axis=1)
    else:
        out = agg + scale * x
    t = _dot(out, W1[...]) + b1[...]
    m = jnp.mean(t, axis=0, keepdims=True)
    c = t - m
    v = jnp.mean(c * c, axis=0, keepdims=True)
    h = c * _rsqrt(v + 1e-5) * g1[...] + be1[...]
    h = jnp.maximum(h, 0.0)
    return _dot(h, W2[...]) + b2[...]


def _tc_gin_body(cat, agg, x_ref, eps, W1, b1, g1, be1, W2, b2, o_ref):
    o_ref[...] = _gin(agg, x_ref[...], eps, W1, b1, g1, be1, W2, b2, cat)


def _tc_gin_addrelu_body(cat, agg, x_ref, prev_ref, eps, W1, b1, g1, be1, W2,
                         b2, o_ref):
    h = _gin(agg, x_ref[...], eps, W1, b1, g1, be1, W2, b2, cat)
    o_ref[...] = jnp.maximum(h + prev_ref[...], 0.0)


def _tc_gin_relu_body(cat, agg, x_ref, eps, W1, b1, g1, be1, W2, b2, o_ref):
    o_ref[...] = jnp.maximum(
        _gin(agg, x_ref[...], eps, W1, b1, g1, be1, W2, b2, cat), 0.0)


def _tc_final_body(agg, x_ref, prev_ref, eps, W1, b1, g1, be1, W2, b2,
                   Wp1, bp1, gp, bep, ap, Wp2, bp2, o_ref):
    h = _gin(agg, x_ref[...], eps, W1, b1, g1, be1, W2, b2, False)
    x2 = jnp.maximum(h + prev_ref[...], 0.0)
    t = _dot(x2, Wp1[...]) + bp1[...]
    m = jnp.mean(t, axis=0, keepdims=True)
    c = t - m
    v = jnp.mean(c * c, axis=0, keepdims=True)
    hb = c * _rsqrt(v + 1e-5) * gp[...] + bep[...]
    a = ap[0, 0]
    hb = jnp.where(hb >= 0, hb, a * hb)
    o = _dot(hb, Wp2[...]) + bp2[...]
    o_ref[...] = jnp.maximum(o, 0.0)


def _f32(shape):
    return jax.ShapeDtypeStruct(shape, jnp.float32)


_tc_gin_cat = pl.pallas_call(functools.partial(_tc_gin_body, True),
                             out_shape=_f32((N, D)))
_tc_gin_addrelu_cat = pl.pallas_call(
    functools.partial(_tc_gin_addrelu_body, True), out_shape=_f32((N, D)))
_tc_gin_relu_cat = pl.pallas_call(functools.partial(_tc_gin_relu_body, True),
                                  out_shape=_f32((N, D)))
_tc_gin_sum = pl.pallas_call(functools.partial(_tc_gin_body, False),
                             out_shape=_f32((N, D)))
_tc_final = pl.pallas_call(_tc_final_body, out_shape=_f32((N, 1)))


def _prep(p):
    return (p["eps"].reshape(1, 1), p["W1"], p["b1"].reshape(1, -1),
            p["g1"].reshape(1, -1), p["be1"].reshape(1, -1), p["W2"],
            p["b2"].reshape(1, -1))


def kernel(x_path, x_link, x_node, params, e_pu, e_ip, e_cn, e_hl):
    zeros = jnp.zeros((NPAD, D), jnp.float32)
    srcs = jnp.stack([e_pu[0], e_hl[0], e_ip[0], e_cn[0]]).reshape(
        4, NW, EPW)
    dsts = jnp.stack([e_pu[1], e_hl[1], e_ip[1], e_cn[1]]).reshape(
        4, NW, NCH, CHUNK)

    agg_pu, agg_hl, agg_ip, agg_cn = _sc_agg_l0(
        x_path, x_node, x_link, srcs, dsts, zeros)

    l0 = params["layer0"]
    x_path1 = _tc_gin_relu_cat(agg_ip, x_path, *_prep(l0["ip"]))
    x_node1 = _tc_gin_relu_cat(agg_cn, x_node, *_prep(l0["cn"]))
    t_pu = _tc_gin_cat(agg_pu, x_link, *_prep(l0["pu"]))
    x_link1 = _tc_gin_addrelu_cat(agg_hl, x_link, t_pu, *_prep(l0["hl"]))

    agg2_pu, agg2_hl = _sc_agg_l1(x_path1, x_node1, srcs[:2], dsts[:2], zeros)

    l1 = params["layer1"]
    pp = params["post"]
    t2_pu = _tc_gin_sum(agg2_pu, x_link1, *_prep(l1["pu"]))
    return _tc_final(agg2_hl, x_link1, t2_pu, *_prep(l1["hl"]),
                     pp["W1"], pp["b1"].reshape(1, -1), pp["g"].reshape(1, -1),
                     pp["be"].reshape(1, -1), pp["a"].reshape(1, 1),
                     pp["W2"], pp["b2"].reshape(1, 1))
